# baseline (speedup 1.0000x reference)
import functools

import jax
import jax.numpy as jnp
from jax import lax
from jax.experimental import pallas as pl
from jax.experimental.pallas import tpu as pltpu
from jax.experimental.pallas import tpu_sc as plsc

B, S, D = 512, 50, 64
NMOD = 14
NSLOT = 26
NOUT = 27
PAIRS = B * S

NC, NS = 2, 16
NW = NC * NS
PPW = PAIRS // NW
CP = 16
NCH = PPW // CP
ROWS = CP * NOUT
NG = ROWS // 16
INW = CP * NMOD * D
MASKROW = CP * NMOD


def _body(tb, ri, mt, out, idx_v, in_v, out_v, mt_v, sem):
    wid = lax.axis_index("s") * NC + lax.axis_index("c")
    iota = lax.iota(jnp.int32, 16)

    pltpu.sync_copy(mt, mt_v)
    for k in range(4):
        in_v[pl.ds(MASKROW * D + k * 16, 16)] = mt_v[pl.ds(k * 16, 16)]

    def chunk_body(c, carry):
        pair0 = pl.multiple_of(wid * PPW + c * CP, CP)
        pltpu.sync_copy(ri.at[pl.ds(pair0 * NSLOT, CP * NSLOT)], idx_v)
        pltpu.sync_copy(
            tb.at[pl.ds(pair0 * NMOD * D, INW)], in_v.at[pl.ds(0, INW)]
        )

        def group_body(g, gc):
            r = iota + g * 16
            p_rel = lax.div(r, NOUT)
            n = r - p_rel * NOUT
            off = jnp.maximum(p_rel * NSLOT + n - 1, 0)
            idx = plsc.load_gather(idx_v, [off])
            keep = idx < (NMOD - 1)
            base = p_rel * NMOD
            src_rel = jnp.where(
                n == 0, base, jnp.where(keep, base + 1 + idx, MASKROW)
            )
            srcf = src_rel * D
            dstf = g * 16 * D + iota * D
            for d in range(D):
                val = plsc.load_gather(in_v, [srcf + d])
                plsc.store_scatter(out_v, [dstf + d], val)
            return gc
        lax.fori_loop(0, NG, group_body, 0)

        pltpu.sync_copy(
            out_v, out.at[pl.ds(pair0 * NOUT * D, ROWS * D)]
        )
        return carry

    lax.fori_loop(0, NCH, chunk_body, 0)


_revert = functools.partial(
    pl.kernel,
    _body,
    out_type=jax.ShapeDtypeStruct((PAIRS * NOUT * D,), jnp.float32),
    mesh=plsc.VectorSubcoreMesh(core_axis_name="c", subcore_axis_name="s"),
    compiler_params=pltpu.CompilerParams(needs_layout_passes=False),
    scratch_types=[
        pltpu.VMEM((CP * NSLOT,), jnp.int32),
        pltpu.VMEM((INW + D,), jnp.float32),
        pltpu.VMEM((ROWS * D,), jnp.float32),
        pltpu.VMEM((D,), jnp.float32),
        pltpu.SemaphoreType.DMA,
    ],
)


def kernel(temporal_block, revert_idx, mask_token):
    tb = temporal_block.reshape(PAIRS * NMOD * D)
    ri = revert_idx.reshape(PAIRS * NSLOT)
    out = _revert()(tb, ri, mask_token)
    return out.reshape(B, S, NOUT, D)


# speedup vs baseline: 1.9796x; 1.9796x over previous
import functools

import jax
import jax.numpy as jnp
from jax import lax
from jax.experimental import pallas as pl
from jax.experimental.pallas import tpu as pltpu
from jax.experimental.pallas import tpu_sc as plsc

B, S, D = 512, 50, 64
NMOD = 14
NSLOT = 26
NOUT = 27
PAIRS = B * S

NC, NS = 2, 16
NW = NC * NS
PPW = PAIRS // NW
CP = 16
NCH = PPW // CP
ROWS = CP * NOUT
NG = ROWS // 16
INW = CP * NMOD * D
MASKROW = CP * NMOD


def _body(tb, ri, mt, out, idx_v, in_v, out_v, mt_v, sem):
    wid = lax.axis_index("s") * NC + lax.axis_index("c")
    iota = lax.iota(jnp.int32, 16)

    pltpu.sync_copy(mt, mt_v)
    for k in range(4):
        in_v[pl.ds(MASKROW * D + k * 16, 16)] = mt_v[pl.ds(k * 16, 16)]

    def chunk_body(c, carry):
        pair0 = pl.multiple_of(wid * PPW + c * CP, CP)
        pltpu.sync_copy(ri.at[pl.ds(pair0 * NSLOT, CP * NSLOT)], idx_v)
        pltpu.sync_copy(
            tb.at[pl.ds(pair0 * NMOD * D, INW)], in_v.at[pl.ds(0, INW)]
        )

        def group_body(g, gc):
            r = iota + g * 16
            p_rel = lax.div(r, NOUT)
            n = r - p_rel * NOUT
            off = jnp.maximum(p_rel * NSLOT + n - 1, 0)
            idx = plsc.load_gather(idx_v, [off])
            keep = idx < (NMOD - 1)
            base = p_rel * NMOD
            src_rel = jnp.where(
                n == 0, base, jnp.where(keep, base + 1 + idx, MASKROW)
            )
            srcf = src_rel * D
            dstf = g * 16 * D + iota * D
            for d in range(D):
                col = (iota + d) & (D - 1)
                val = plsc.load_gather(in_v, [srcf + col])
                plsc.store_scatter(out_v, [dstf + col], val)
            return gc
        lax.fori_loop(0, NG, group_body, 0)

        pltpu.sync_copy(
            out_v, out.at[pl.ds(pair0 * NOUT * D, ROWS * D)]
        )
        return carry

    lax.fori_loop(0, NCH, chunk_body, 0)


_revert = functools.partial(
    pl.kernel,
    _body,
    out_type=jax.ShapeDtypeStruct((PAIRS * NOUT * D,), jnp.float32),
    mesh=plsc.VectorSubcoreMesh(core_axis_name="c", subcore_axis_name="s"),
    compiler_params=pltpu.CompilerParams(needs_layout_passes=False),
    scratch_types=[
        pltpu.VMEM((CP * NSLOT,), jnp.int32),
        pltpu.VMEM((INW + D,), jnp.float32),
        pltpu.VMEM((ROWS * D,), jnp.float32),
        pltpu.VMEM((D,), jnp.float32),
        pltpu.SemaphoreType.DMA,
    ],
)


def kernel(temporal_block, revert_idx, mask_token):
    tb = temporal_block.reshape(PAIRS * NMOD * D)
    ri = revert_idx.reshape(PAIRS * NSLOT)
    out = _revert()(tb, ri, mask_token)
    return out.reshape(B, S, NOUT, D)


# speedup vs baseline: 2.5092x; 1.2675x over previous
import functools

import jax
import jax.numpy as jnp
from jax import lax
from jax.experimental import pallas as pl
from jax.experimental.pallas import tpu as pltpu
from jax.experimental.pallas import tpu_sc as plsc

B, S, D = 512, 50, 64
NMOD = 14
NSLOT = 26
NOUT = 27
PAIRS = B * S
PSUB, PLANE = 32, 128
PBLK = PSUB * PLANE

NC, NS = 2, 16
NW = NC * NS
PPW = PAIRS // NW
CP = 16
NCH = PPW // CP
ROWS = CP * NOUT
NG = ROWS // 16
INW = CP * NMOD * D
MASKROW = CP * NMOD


def _body(tb, ri, mt, out, idx_v, in_v, out_v, mt_v, sem):
    wid = lax.axis_index("s") * NC + lax.axis_index("c")
    iota = lax.iota(jnp.int32, 16)

    pltpu.sync_copy(mt, mt_v)
    for k in range(4):
        in_v[pl.ds(MASKROW * D + k * 16, 16)] = mt_v[pl.ds(k * 16, 16)]

    def chunk_body(c, carry):
        pair0 = pl.multiple_of(wid * PPW + c * CP, CP)
        pltpu.sync_copy(ri.at[pl.ds(pair0 * NSLOT, CP * NSLOT)], idx_v)
        pltpu.sync_copy(
            tb.at[pl.ds(pair0 * NMOD * D, INW)], in_v.at[pl.ds(0, INW)]
        )

        def group_body(g, gc):
            r = iota + g * 16
            p_rel = lax.div(r, NOUT)
            n = r - p_rel * NOUT
            off = jnp.maximum(p_rel * NSLOT + n - 1, 0)
            idx = plsc.load_gather(idx_v, [off])
            keep = idx < (NMOD - 1)
            base = p_rel * NMOD
            src_rel = jnp.where(
                n == 0, base, jnp.where(keep, base + 1 + idx, MASKROW)
            )
            srcf = src_rel * D
            dstf = p_rel * PBLK + n * PLANE
            for d in range(D):
                col = (iota + d) & (D - 1)
                val = plsc.load_gather(in_v, [srcf + col])
                plsc.store_scatter(out_v, [dstf + col], val)
            return gc
        lax.fori_loop(0, NG, group_body, 0)

        pltpu.sync_copy(out_v, out.at[pl.ds(pair0 * PBLK, CP * PBLK)])
        return carry

    lax.fori_loop(0, NCH, chunk_body, 0)


_revert = functools.partial(
    pl.kernel,
    _body,
    out_type=jax.ShapeDtypeStruct((PAIRS * PBLK,), jnp.float32),
    mesh=plsc.VectorSubcoreMesh(core_axis_name="c", subcore_axis_name="s"),
    compiler_params=pltpu.CompilerParams(needs_layout_passes=False),
    scratch_types=[
        pltpu.VMEM((CP * NSLOT,), jnp.int32),
        pltpu.VMEM((INW + D,), jnp.float32),
        pltpu.VMEM((CP * PBLK,), jnp.float32),
        pltpu.VMEM((D,), jnp.float32),
        pltpu.SemaphoreType.DMA,
    ],
)


def kernel(temporal_block, revert_idx, mask_token):
    tb = temporal_block.reshape(PAIRS * NMOD * D)
    ri = revert_idx.reshape(PAIRS * NSLOT)
    out = _revert()(tb, ri, mask_token)
    out = out.reshape(B, S, PSUB, PLANE)
    return out[:, :, :NOUT, :D]


# speedup vs baseline: 3.1824x; 1.2683x over previous
import functools

import jax
import jax.numpy as jnp
from jax import lax
from jax.experimental import pallas as pl
from jax.experimental.pallas import tpu as pltpu
from jax.experimental.pallas import tpu_sc as plsc

B, S, D = 512, 50, 64
NMOD = 14
NSLOT = 26
NOUT = 27
PAIRS = B * S
PSUB, PLANE = 32, 128
PBLK = PSUB * PLANE

NC, NS = 2, 16
NW = NC * NS
PPW = PAIRS // NW
CP = 8
NCH = PPW // CP
ROWS = CP * NOUT
NG = (ROWS + 15) // 16
INW = CP * NMOD * D
IDXW = CP * NSLOT
MASKROW = CP * NMOD


def _body(tb, ri, mt, out,
          idx_a, idx_b, in_a, in_b, out_a, out_b, mt_v,
          sem_in_a, sem_in_b, sem_out_a, sem_out_b):
    wid = lax.axis_index("s") * NC + lax.axis_index("c")
    iota = lax.iota(jnp.int32, 16)

    def fire_in(c, in_v, idx_v, sem):
        pair0 = pl.multiple_of(wid * PPW + c * CP, CP)
        pltpu.make_async_copy(
            ri.at[pl.ds(pair0 * NSLOT, IDXW)], idx_v, sem).start()
        pltpu.make_async_copy(
            tb.at[pl.ds(pair0 * NMOD * D, INW)], in_v.at[pl.ds(0, INW)], sem
        ).start()

    def wait_in(in_v, idx_v, sem):
        pltpu.make_async_copy(ri.at[pl.ds(0, IDXW)], idx_v, sem).wait()
        pltpu.make_async_copy(
            tb.at[pl.ds(0, INW)], in_v.at[pl.ds(0, INW)], sem).wait()

    def fire_out(c, out_v, sem):
        pair0 = pl.multiple_of(wid * PPW + c * CP, CP)
        pltpu.make_async_copy(
            out_v, out.at[pl.ds(pair0 * PBLK, CP * PBLK)], sem).start()

    def drain_out(out_v, sem):
        pltpu.make_async_copy(
            out_v, out.at[pl.ds(0, CP * PBLK)], sem).wait()

    def compute(in_v, idx_v, out_v):
        def group_body(g, gc):
            rowid = iota + g * 16
            rid = jnp.minimum(rowid, ROWS - 1)
            p_rel = lax.div(rid, NOUT)
            n = rid - p_rel * NOUT
            off = jnp.maximum(p_rel * NSLOT + n - 1, 0)
            idx = plsc.load_gather(idx_v, [off])
            keep = idx < (NMOD - 1)
            base = p_rel * NMOD
            src_rel = jnp.where(
                n == 0, base, jnp.where(keep, base + 1 + idx, MASKROW)
            )
            srcf = src_rel * D
            dstf = p_rel * PBLK + n * PLANE
            valid = rowid < ROWS
            for d in range(D):
                col = (iota + d) & (D - 1)
                val = plsc.load_gather(in_v, [srcf + col])
                plsc.store_scatter(out_v, [dstf + col], val, mask=valid)
            return gc
        lax.fori_loop(0, NG, group_body, 0)

    pltpu.sync_copy(mt, mt_v)
    for k in range(4):
        mtk = mt_v[pl.ds(k * 16, 16)]
        in_a[pl.ds(MASKROW * D + k * 16, 16)] = mtk
        in_b[pl.ds(MASKROW * D + k * 16, 16)] = mtk

    fire_in(0, in_a, idx_a, sem_in_a)
    fire_in(1, in_b, idx_b, sem_in_b)

    def round_body(cc, carry):
        ca = cc * 2
        cb = ca + 1
        wait_in(in_a, idx_a, sem_in_a)

        @pl.when(cc > 0)
        def _():
            drain_out(out_a, sem_out_a)
        compute(in_a, idx_a, out_a)
        fire_out(ca, out_a, sem_out_a)

        @pl.when(cc < NCH // 2 - 1)
        def _():
            fire_in(ca + 2, in_a, idx_a, sem_in_a)

        wait_in(in_b, idx_b, sem_in_b)

        @pl.when(cc > 0)
        def _():
            drain_out(out_b, sem_out_b)
        compute(in_b, idx_b, out_b)
        fire_out(cb, out_b, sem_out_b)

        @pl.when(cc < NCH // 2 - 1)
        def _():
            fire_in(cb + 2, in_b, idx_b, sem_in_b)
        return carry

    lax.fori_loop(0, NCH // 2, round_body, 0)
    drain_out(out_a, sem_out_a)
    drain_out(out_b, sem_out_b)


_revert = functools.partial(
    pl.kernel,
    _body,
    out_type=jax.ShapeDtypeStruct((PAIRS * PBLK,), jnp.float32),
    mesh=plsc.VectorSubcoreMesh(core_axis_name="c", subcore_axis_name="s"),
    compiler_params=pltpu.CompilerParams(needs_layout_passes=False),
    scratch_types=[
        pltpu.VMEM((IDXW,), jnp.int32),
        pltpu.VMEM((IDXW,), jnp.int32),
        pltpu.VMEM((INW + D,), jnp.float32),
        pltpu.VMEM((INW + D,), jnp.float32),
        pltpu.VMEM((CP * PBLK,), jnp.float32),
        pltpu.VMEM((CP * PBLK,), jnp.float32),
        pltpu.VMEM((D,), jnp.float32),
        pltpu.SemaphoreType.DMA,
        pltpu.SemaphoreType.DMA,
        pltpu.SemaphoreType.DMA,
        pltpu.SemaphoreType.DMA,
    ],
)


def kernel(temporal_block, revert_idx, mask_token):
    tb = temporal_block.reshape(PAIRS * NMOD * D)
    ri = revert_idx.reshape(PAIRS * NSLOT)
    out = _revert()(tb, ri, mask_token)
    out = out.reshape(B, S, PSUB, PLANE)
    return out[:, :, :NOUT, :D]
